# baseline (speedup 1.0000x reference)
import functools

import jax
import jax.numpy as jnp
from jax import lax
from jax.experimental import pallas as pl
from jax.experimental.pallas import tpu as pltpu
from jax.experimental.pallas import tpu_sc as plsc

HD = 256
T_ROWS = 64
NW = 32
NREP = 2
BLK = 96
NBUF = 4


def _table_body(w0, w1, w2, o):
    for i in range(5):
        for j in range(6):
            for k in range(2):
                r = 12 * i + 2 * j + k
                o[pl.ds(r, 1), :] = (
                    w0[pl.ds(i, 1), :] + w1[pl.ds(j, 1), :] + w2[pl.ds(k, 1), :]
                )
    for r in range(60, T_ROWS):
        o[pl.ds(r, 1), :] = jnp.zeros((1, HD), jnp.float32)


def _build_table(W0, W1, W2):
    nrep = NW * NREP
    return pl.pallas_call(
        _table_body,
        grid=(nrep,),
        in_specs=[
            pl.BlockSpec(W0.shape, lambda i: (0, 0)),
            pl.BlockSpec(W1.shape, lambda i: (0, 0)),
            pl.BlockSpec(W2.shape, lambda i: (0, 0)),
        ],
        out_specs=pl.BlockSpec((T_ROWS, HD), lambda i: (i, 0)),
        out_shape=jax.ShapeDtypeStruct((nrep * T_ROWS, HD), jnp.float32),
    )(W0, W1, W2)


def _sc_body(chunk, a0_hbm, a1_hbm, a2_hbm, t_hbm, out_hbm,
             a0_v, a1_v, a2_v, cidx_v,
             b0, b1, b2, b3, g0, g1, g2, g3, s0, s1, s2, s3):
    bufs = (b0, b1, b2, b3)
    gs = (g0, g1, g2, g3)
    ss = (s0, s1, s2, s3)
    wid = lax.axis_index("s") * 2 + lax.axis_index("c")
    base = wid * chunk
    pltpu.sync_copy(a0_hbm.at[pl.ds(base, chunk)], a0_v.at[pl.ds(0, chunk)])
    pltpu.sync_copy(a1_hbm.at[pl.ds(base, chunk)], a1_v.at[pl.ds(0, chunk)])
    pltpu.sync_copy(a2_hbm.at[pl.ds(base, chunk)], a2_v.at[pl.ds(0, chunk)])

    nvec = (chunk + 15) // 16
    rep0 = wid * NREP * T_ROWS

    def cbody(i, _):
        s = i * 16
        c = a0_v[pl.ds(s, 16)] * 12 + a1_v[pl.ds(s, 16)] * 2 + a2_v[pl.ds(s, 16)]
        c = jnp.minimum(jnp.maximum(c, 0), T_ROWS - 1)
        c = c + (rep0 + (i & (NREP - 1)) * T_ROWS)
        cidx_v[pl.ds(s, 16)] = c
        return 0

    lax.fori_loop(0, nvec, cbody, 0)

    nfull = chunk // BLK
    tail = chunk - nfull * BLK
    assert nfull % NBUF == 0 and nfull // NBUF >= 2

    def start_gather(j, b):
        pltpu.async_copy(t_hbm.at[cidx_v.at[pl.ds(j * BLK, BLK)]], bufs[b], gs[b])

    def wait_gather(b):
        pltpu.make_async_copy(out_hbm.at[pl.ds(0, BLK)], bufs[b], gs[b]).wait()

    def start_store(j, b):
        pltpu.async_copy(bufs[b], out_hbm.at[pl.ds(base + j * BLK, BLK)], ss[b])

    def wait_store(b):
        pltpu.make_async_copy(bufs[b], out_hbm.at[pl.ds(0, BLK)], ss[b]).wait()

    start_gather(0, 0)
    for j in range(1, NBUF):
        wait_gather(j - 1)
        start_store(j - 1, j - 1)
        start_gather(j, j)

    def pbody(j2, _):
        j0 = j2 * NBUF
        for b in range(NBUF):
            j = j0 + b
            bp = (b + NBUF - 1) % NBUF
            wait_gather(bp)
            start_store(j - 1, bp)
            wait_store(b)
            start_gather(j, b)
        return 0

    lax.fori_loop(1, nfull // NBUF, pbody, 0)

    last_b = (nfull - 1) % NBUF
    wait_gather(last_b)
    start_store(nfull - 1, last_b)
    if tail:
        s = nfull * BLK
        wait_store(0)
        pltpu.async_copy(
            t_hbm.at[cidx_v.at[pl.ds(s, tail)]], b0.at[pl.ds(0, tail)], g0
        ).wait()
        pltpu.sync_copy(b0.at[pl.ds(0, tail)], out_hbm.at[pl.ds(base + s, tail)])
        for b in range(1, NBUF):
            wait_store(b)
    else:
        for b in range(NBUF):
            wait_store(b)


def _sc_gather(a0, a1, a2, T):
    E = a0.shape[0]
    assert E % NW == 0
    chunk = E // NW
    assert chunk % 8 == 0
    chunk_pad = ((chunk + 15) // 16) * 16
    mesh = plsc.VectorSubcoreMesh(core_axis_name="c", subcore_axis_name="s")
    kfn = pl.kernel(
        functools.partial(_sc_body, chunk),
        mesh=mesh,
        out_type=jax.ShapeDtypeStruct((E, HD), jnp.float32),
        scratch_types=(
            [pltpu.VMEM((chunk_pad,), jnp.int32)] * 4
            + [pltpu.VMEM((BLK, HD), jnp.float32)] * NBUF
            + [pltpu.SemaphoreType.DMA] * (2 * NBUF)
        ),
    )
    return kfn(a0, a1, a2, T)


def kernel(edge_attr, W0, W1, W2):
    T = _build_table(W0, W1, W2)
    a = edge_attr.astype(jnp.int32)
    return _sc_gather(a[:, 0], a[:, 1], a[:, 2], T)


# speedup vs baseline: 1.0049x; 1.0049x over previous
import functools

import jax
import jax.numpy as jnp
from jax import lax
from jax.experimental import pallas as pl
from jax.experimental.pallas import tpu as pltpu
from jax.experimental.pallas import tpu_sc as plsc

HD = 256
T_ROWS = 64
NW = 32
NREP = 4
BLK = 96
NBUF = 4


def _table_body(w0, w1, w2, o):
    for i in range(5):
        for j in range(6):
            for k in range(2):
                r = 12 * i + 2 * j + k
                o[pl.ds(r, 1), :] = (
                    w0[pl.ds(i, 1), :] + w1[pl.ds(j, 1), :] + w2[pl.ds(k, 1), :]
                )
    for r in range(60, T_ROWS):
        o[pl.ds(r, 1), :] = jnp.zeros((1, HD), jnp.float32)


def _build_table(W0, W1, W2):
    nrep = NW * NREP
    return pl.pallas_call(
        _table_body,
        grid=(nrep,),
        in_specs=[
            pl.BlockSpec(W0.shape, lambda i: (0, 0)),
            pl.BlockSpec(W1.shape, lambda i: (0, 0)),
            pl.BlockSpec(W2.shape, lambda i: (0, 0)),
        ],
        out_specs=pl.BlockSpec((T_ROWS, HD), lambda i: (i, 0)),
        out_shape=jax.ShapeDtypeStruct((nrep * T_ROWS, HD), jnp.float32),
    )(W0, W1, W2)


def _sc_body(chunk, a0_hbm, a1_hbm, a2_hbm, t_hbm, out_hbm,
             a0_v, a1_v, a2_v, cidx_v,
             b0, b1, b2, b3, g0, g1, g2, g3, s0, s1, s2, s3):
    bufs = (b0, b1, b2, b3)
    gs = (g0, g1, g2, g3)
    ss = (s0, s1, s2, s3)
    wid = lax.axis_index("s") * 2 + lax.axis_index("c")
    base = wid * chunk
    pltpu.sync_copy(a0_hbm.at[pl.ds(base, chunk)], a0_v.at[pl.ds(0, chunk)])
    pltpu.sync_copy(a1_hbm.at[pl.ds(base, chunk)], a1_v.at[pl.ds(0, chunk)])
    pltpu.sync_copy(a2_hbm.at[pl.ds(base, chunk)], a2_v.at[pl.ds(0, chunk)])

    nvec = (chunk + 15) // 16
    rep0 = wid * NREP * T_ROWS

    def cbody(i, _):
        s = i * 16
        c = a0_v[pl.ds(s, 16)] * 12 + a1_v[pl.ds(s, 16)] * 2 + a2_v[pl.ds(s, 16)]
        c = jnp.minimum(jnp.maximum(c, 0), T_ROWS - 1)
        c = c + (rep0 + (i & (NREP - 1)) * T_ROWS)
        cidx_v[pl.ds(s, 16)] = c
        return 0

    lax.fori_loop(0, nvec, cbody, 0)

    nfull = chunk // BLK
    tail = chunk - nfull * BLK
    assert nfull % NBUF == 0 and nfull // NBUF >= 2

    def start_gather(j, b):
        pltpu.async_copy(t_hbm.at[cidx_v.at[pl.ds(j * BLK, BLK)]], bufs[b], gs[b])

    def wait_gather(b):
        pltpu.make_async_copy(out_hbm.at[pl.ds(0, BLK)], bufs[b], gs[b]).wait()

    def start_store(j, b):
        pltpu.async_copy(bufs[b], out_hbm.at[pl.ds(base + j * BLK, BLK)], ss[b])

    def wait_store(b):
        pltpu.make_async_copy(bufs[b], out_hbm.at[pl.ds(0, BLK)], ss[b]).wait()

    start_gather(0, 0)
    start_gather(1, 1)
    for j in range(2, NBUF):
        wait_gather(j - 2)
        start_store(j - 2, j - 2)
        start_gather(j, j)

    def pbody(j2, _):
        j0 = j2 * NBUF
        for b in range(NBUF):
            j = j0 + b
            bp = (b + NBUF - 2) % NBUF
            wait_gather(bp)
            start_store(j - 2, bp)
            wait_store(b)
            start_gather(j, b)
        return 0

    lax.fori_loop(1, nfull // NBUF, pbody, 0)

    for j in (nfull - 2, nfull - 1):
        bl = j % NBUF
        wait_gather(bl)
        start_store(j, bl)
    if tail:
        s = nfull * BLK
        wait_store(0)
        pltpu.async_copy(
            t_hbm.at[cidx_v.at[pl.ds(s, tail)]], b0.at[pl.ds(0, tail)], g0
        ).wait()
        pltpu.sync_copy(b0.at[pl.ds(0, tail)], out_hbm.at[pl.ds(base + s, tail)])
        for b in range(1, NBUF):
            wait_store(b)
    else:
        for b in range(NBUF):
            wait_store(b)


def _sc_gather(a0, a1, a2, T):
    E = a0.shape[0]
    assert E % NW == 0
    chunk = E // NW
    assert chunk % 8 == 0
    chunk_pad = ((chunk + 15) // 16) * 16
    mesh = plsc.VectorSubcoreMesh(core_axis_name="c", subcore_axis_name="s")
    kfn = pl.kernel(
        functools.partial(_sc_body, chunk),
        mesh=mesh,
        out_type=jax.ShapeDtypeStruct((E, HD), jnp.float32),
        scratch_types=(
            [pltpu.VMEM((chunk_pad,), jnp.int32)] * 4
            + [pltpu.VMEM((BLK, HD), jnp.float32)] * NBUF
            + [pltpu.SemaphoreType.DMA] * (2 * NBUF)
        ),
    )
    return kfn(a0, a1, a2, T)


def kernel(edge_attr, W0, W1, W2):
    T = _build_table(W0, W1, W2)
    a = edge_attr.astype(jnp.int32)
    return _sc_gather(a[:, 0], a[:, 1], a[:, 2], T)


# speedup vs baseline: 1.0200x; 1.0150x over previous
import functools

import jax
import jax.numpy as jnp
from jax import lax
from jax.experimental import pallas as pl
from jax.experimental.pallas import tpu as pltpu
from jax.experimental.pallas import tpu_sc as plsc

HD = 256
T_ROWS = 64
NW = 32
NREP = 4
BLK = 96
NBUF = 4


def _table_body(w0, w1, w2, o):
    for i in range(5):
        for j in range(6):
            for k in range(2):
                r = 12 * i + 2 * j + k
                o[pl.ds(r, 1), :] = (
                    w0[pl.ds(i, 1), :] + w1[pl.ds(j, 1), :] + w2[pl.ds(k, 1), :]
                )
    for r in range(60, T_ROWS):
        o[pl.ds(r, 1), :] = jnp.zeros((1, HD), jnp.float32)


def _build_table(W0, W1, W2):
    nrep = NW * NREP
    return pl.pallas_call(
        _table_body,
        grid=(nrep,),
        in_specs=[
            pl.BlockSpec(W0.shape, lambda i: (0, 0)),
            pl.BlockSpec(W1.shape, lambda i: (0, 0)),
            pl.BlockSpec(W2.shape, lambda i: (0, 0)),
        ],
        out_specs=pl.BlockSpec((T_ROWS, HD), lambda i: (i, 0)),
        out_shape=jax.ShapeDtypeStruct((nrep * T_ROWS, HD), jnp.float32),
    )(W0, W1, W2)


def _sc_body(chunk, a0_hbm, a1_hbm, a2_hbm, t_hbm, out_hbm,
             a0_v, a1_v, a2_v, cidx_v,
             b0, b1, b2, b3, g0, g1, g2, g3, s0, s1, s2, s3):
    bufs = (b0, b1, b2, b3)
    gs = (g0, g1, g2, g3)
    ss = (s0, s1, s2, s3)
    wid = lax.axis_index("s") * 2 + lax.axis_index("c")
    base = wid * chunk
    pltpu.sync_copy(a0_hbm.at[pl.ds(base, chunk)], a0_v.at[pl.ds(0, chunk)])
    pltpu.sync_copy(a1_hbm.at[pl.ds(base, chunk)], a1_v.at[pl.ds(0, chunk)])
    pltpu.sync_copy(a2_hbm.at[pl.ds(base, chunk)], a2_v.at[pl.ds(0, chunk)])

    nvec = (chunk + 15) // 16
    rep0 = wid * NREP * T_ROWS

    def cbody(i, _):
        s = i * 16
        c = a0_v[pl.ds(s, 16)] * 12 + a1_v[pl.ds(s, 16)] * 2 + a2_v[pl.ds(s, 16)]
        c = jnp.minimum(jnp.maximum(c, 0), T_ROWS - 1)
        c = c + (rep0 + (i & (NREP - 1)) * T_ROWS)
        cidx_v[pl.ds(s, 16)] = c
        return 0

    lax.fori_loop(0, nvec, cbody, 0)

    nfull = chunk // BLK
    tail = chunk - nfull * BLK
    assert nfull % NBUF == 0 and nfull // NBUF >= 2

    def start_gather(j, b):
        pltpu.async_copy(t_hbm.at[cidx_v.at[pl.ds(j * BLK, BLK)]], bufs[b], gs[b])

    def wait_gather(b):
        pltpu.make_async_copy(out_hbm.at[pl.ds(0, BLK)], bufs[b], gs[b]).wait()

    def start_store(j, b):
        pltpu.async_copy(bufs[b], out_hbm.at[pl.ds(base + j * BLK, BLK)], ss[b])

    def wait_store(b):
        pltpu.make_async_copy(bufs[b], out_hbm.at[pl.ds(0, BLK)], ss[b]).wait()

    start_gather(0, 0)
    for j in range(1, NBUF):
        wait_gather(j - 1)
        start_store(j - 1, j - 1)
        start_gather(j, j)

    def pbody(j2, _):
        j0 = j2 * NBUF
        for b in range(NBUF):
            j = j0 + b
            bp = (b + NBUF - 1) % NBUF
            wait_gather(bp)
            start_store(j - 1, bp)
            wait_store(b)
            start_gather(j, b)
        return 0

    lax.fori_loop(1, nfull // NBUF, pbody, 0)

    last_b = (nfull - 1) % NBUF
    wait_gather(last_b)
    start_store(nfull - 1, last_b)
    if tail:
        s = nfull * BLK
        wait_store(0)
        pltpu.async_copy(
            t_hbm.at[cidx_v.at[pl.ds(s, tail)]], b0.at[pl.ds(0, tail)], g0
        ).wait()
        pltpu.sync_copy(b0.at[pl.ds(0, tail)], out_hbm.at[pl.ds(base + s, tail)])
        for b in range(1, NBUF):
            wait_store(b)
    else:
        for b in range(NBUF):
            wait_store(b)


def _sc_gather(a0, a1, a2, T):
    E = a0.shape[0]
    assert E % NW == 0
    chunk = E // NW
    assert chunk % 8 == 0
    chunk_pad = ((chunk + 15) // 16) * 16
    mesh = plsc.VectorSubcoreMesh(core_axis_name="c", subcore_axis_name="s")
    kfn = pl.kernel(
        functools.partial(_sc_body, chunk),
        mesh=mesh,
        out_type=jax.ShapeDtypeStruct((E, HD), jnp.float32),
        scratch_types=(
            [pltpu.VMEM((chunk_pad,), jnp.int32)] * 4
            + [pltpu.VMEM((BLK, HD), jnp.float32)] * NBUF
            + [pltpu.SemaphoreType.DMA] * (2 * NBUF)
        ),
    )
    return kfn(a0, a1, a2, T)


def kernel(edge_attr, W0, W1, W2):
    T = _build_table(W0, W1, W2)
    a = edge_attr.astype(jnp.int32)
    return _sc_gather(a[:, 0], a[:, 1], a[:, 2], T)
